# baseline (speedup 1.0000x reference)
import functools

import jax
import jax.numpy as jnp
from jax import lax
from jax.experimental import pallas as pl
from jax.experimental.pallas import tpu as pltpu
from jax.experimental.pallas import tpu_sc as plsc

_B = 16384
_BLK = 2048
_NB = _B // _BLK

_NC, _NS, _L = 2, 16, 16
_NW = _NC * _NS
_BPW = _B // _NW
_C = 128
_NCH = _BPW // _C


def _prep_body(tmp_ref, str_ref, st_ref, kt_ref, tt_ref,
               tw_ref, tb_ref, sw_ref, sb_ref, fw_ref, fb_ref,
               d_ref, sp_ref, kt3_ref):
    f32 = jnp.float32
    fw = fw_ref[...]
    Ws = fw[:, 0:32]
    Wk = fw[:, 32:48]
    Wt = fw[:, 48:64]
    Wtem = fw[:, 64:80]
    Wstr = fw[:, 80:112]

    sp_ref[...] = jnp.dot(st_ref[...], Ws.T, preferred_element_type=f32)
    KP = jnp.dot(kt_ref[...], Wk.T, preferred_element_type=f32)
    TP = jnp.dot(tt_ref[...], Wt.T, preferred_element_type=f32)
    kt3_ref[...] = KP[:, None, :] + TP[None, :, :]

    tW = jnp.dot(tw_ref[...].T, Wtem.T, preferred_element_type=f32)
    sW = jnp.dot(sw_ref[...].T, Wstr.T, preferred_element_type=f32)
    bias = (fb_ref[...]
            + jnp.dot(tb_ref[...], Wtem.T, preferred_element_type=f32)
            + jnp.dot(sb_ref[...], Wstr.T, preferred_element_type=f32))
    d_ref[...] = (jnp.dot(tmp_ref[0], tW, preferred_element_type=f32)
                  + jnp.dot(str_ref[0], sW, preferred_element_type=f32)
                  + bias)


def _prep(tempos, structures, style_table, key_table, timefeel_table,
          tempo_W, tempo_b, structure_W, structure_b, final_W, final_b):
    tmp3 = tempos.reshape(_NB, _BLK, 1)
    str3 = structures.reshape(_NB, _BLK, 10)
    tb2 = tempo_b.reshape(1, 16)
    sb2 = structure_b.reshape(1, 32)
    fb2 = final_b.reshape(1, 128)
    full = lambda shape: pl.BlockSpec(shape, lambda i: (0,) * len(shape))
    d, sp, kt3 = pl.pallas_call(
        _prep_body,
        grid=(_NB,),
        in_specs=[
            pl.BlockSpec((1, _BLK, 1), lambda i: (i, 0, 0)),
            pl.BlockSpec((1, _BLK, 10), lambda i: (i, 0, 0)),
            full((50, 32)),
            full((24, 16)),
            full((20, 16)),
            full((16, 1)),
            full((1, 16)),
            full((32, 10)),
            full((1, 32)),
            full((128, 112)),
            full((1, 128)),
        ],
        out_specs=[
            pl.BlockSpec((_BLK, 128), lambda i: (i, 0)),
            full((50, 128)),
            full((24, 20, 128)),
        ],
        out_shape=[
            jax.ShapeDtypeStruct((_B, 128), jnp.float32),
            jax.ShapeDtypeStruct((50, 128), jnp.float32),
            jax.ShapeDtypeStruct((24, 20, 128), jnp.float32),
        ],
    )(tmp3, str3, style_table, key_table, timefeel_table,
      tempo_W, tb2, structure_W, sb2, final_W, fb2)
    return d, sp, kt3.reshape(480, 128)


def _sc_body(sid_hbm, kid_hbm, tid_hbm, sp_hbm, kt_hbm, d_hbm, out_hbm,
             sidx_v, kidx_v, tidx_v, fidx_v, srows_v, ktrows_v, acc_v,
             sem_s, sem_k):
    wid = lax.axis_index("s") * _NC + lax.axis_index("c")
    for ch in range(_NCH):
        base = wid * _BPW + ch * _C
        pltpu.sync_copy(sid_hbm.at[pl.ds(base, _C)], sidx_v)
        pltpu.sync_copy(kid_hbm.at[pl.ds(base, _C)], kidx_v)
        pltpu.sync_copy(tid_hbm.at[pl.ds(base, _C)], tidx_v)

        def fuse(i, carry):
            off = i * _L
            fidx_v[pl.ds(off, _L)] = kidx_v[pl.ds(off, _L)] * 20 + tidx_v[pl.ds(off, _L)]
            return carry
        lax.fori_loop(0, _C // _L, fuse, 0)

        g1 = pltpu.async_copy(sp_hbm.at[sidx_v], srows_v, sem_s)
        g2 = pltpu.async_copy(kt_hbm.at[fidx_v], ktrows_v, sem_k)
        pltpu.sync_copy(d_hbm.at[pl.ds(base, _C)], acc_v)
        g1.wait()
        g2.wait()

        def add_row(r, carry):
            for o in range(128 // _L):
                sl = pl.ds(o * _L, _L)
                acc_v[r, sl] = acc_v[r, sl] + srows_v[r, sl] + ktrows_v[r, sl]
            return carry
        lax.fori_loop(0, _C, add_row, 0)

        pltpu.sync_copy(acc_v, out_hbm.at[pl.ds(base, _C)])


@functools.lru_cache(maxsize=1)
def _sc_main():
    return functools.partial(
        pl.kernel,
        out_type=jax.ShapeDtypeStruct((_B, 128), jnp.float32),
        mesh=plsc.VectorSubcoreMesh(core_axis_name="c", subcore_axis_name="s",
                                    num_cores=_NC, num_subcores=_NS),
        scratch_types=[
            pltpu.VMEM((_C,), jnp.int32),
            pltpu.VMEM((_C,), jnp.int32),
            pltpu.VMEM((_C,), jnp.int32),
            pltpu.VMEM((_C,), jnp.int32),
            pltpu.VMEM((_C, 128), jnp.float32),
            pltpu.VMEM((_C, 128), jnp.float32),
            pltpu.VMEM((_C, 128), jnp.float32),
            pltpu.SemaphoreType.DMA,
            pltpu.SemaphoreType.DMA,
        ],
    )(_sc_body)


def kernel(style_ids, key_ids, timefeel_ids, tempos, structures,
           style_table, key_table, timefeel_table,
           tempo_W, tempo_b, structure_W, structure_b,
           final_W, final_b):
    d, sp, kt = _prep(tempos, structures, style_table, key_table,
                      timefeel_table, tempo_W, tempo_b, structure_W,
                      structure_b, final_W, final_b)
    sid = style_ids.astype(jnp.int32)
    kid = key_ids.astype(jnp.int32)
    tid = timefeel_ids.astype(jnp.int32)
    return _sc_main()(sid, kid, tid, sp, kt, d)


# speedup vs baseline: 1.0001x; 1.0001x over previous
import functools

import jax
import jax.numpy as jnp
from jax import lax
from jax.experimental import pallas as pl
from jax.experimental.pallas import tpu as pltpu
from jax.experimental.pallas import tpu_sc as plsc

_B = 16384
_BLK = 2048
_NB = _B // _BLK

_NC, _NS, _L = 2, 16, 16
_NW = _NC * _NS
_BPW = _B // _NW
_C = 128
_NCH = _BPW // _C


def _sc_body(sid_hbm, kid_hbm, tid_hbm, st_hbm, kt_hbm, tt_hbm,
             es_hbm, ek_hbm, et_hbm,
             sidx_v, kidx_v, tidx_v, srows_v, krows_v, trows_v, sem):
    wid = lax.axis_index("s") * _NC + lax.axis_index("c")
    base = wid * _NCH
    pltpu.sync_copy(sid_hbm.at[pl.ds(base, _NCH)], sidx_v)
    pltpu.sync_copy(kid_hbm.at[pl.ds(base, _NCH)], kidx_v)
    pltpu.sync_copy(tid_hbm.at[pl.ds(base, _NCH)], tidx_v)
    copies = []
    for j in range(_NCH):
        dst = pl.ds(j * _C, _C)
        copies.append(pltpu.async_copy(st_hbm.at[sidx_v.at[j]], srows_v.at[dst], sem))
        copies.append(pltpu.async_copy(kt_hbm.at[kidx_v.at[j]], krows_v.at[dst], sem))
        copies.append(pltpu.async_copy(tt_hbm.at[tidx_v.at[j]], trows_v.at[dst], sem))
    for c in copies:
        c.wait()
    out_sl = pl.ds(wid * _BPW, _BPW)
    pltpu.sync_copy(srows_v, es_hbm.at[out_sl])
    pltpu.sync_copy(krows_v, ek_hbm.at[out_sl])
    pltpu.sync_copy(trows_v, et_hbm.at[out_sl])


@functools.lru_cache(maxsize=1)
def _sc_gather():
    return functools.partial(
        pl.kernel,
        out_type=[
            jax.ShapeDtypeStruct((_B, 32), jnp.float32),
            jax.ShapeDtypeStruct((_B, 16), jnp.float32),
            jax.ShapeDtypeStruct((_B, 16), jnp.float32),
        ],
        mesh=plsc.VectorSubcoreMesh(core_axis_name="c", subcore_axis_name="s",
                                    num_cores=_NC, num_subcores=_NS),
        compiler_params=pltpu.CompilerParams(use_tc_tiling_on_sc=False),
        scratch_types=[
            pltpu.VMEM((_NCH, _C), jnp.int32),
            pltpu.VMEM((_NCH, _C), jnp.int32),
            pltpu.VMEM((_NCH, _C), jnp.int32),
            pltpu.VMEM((_BPW, 32), jnp.float32),
            pltpu.VMEM((_BPW, 16), jnp.float32),
            pltpu.VMEM((_BPW, 16), jnp.float32),
            pltpu.SemaphoreType.DMA,
        ],
    )(_sc_body)


def _tc_body(es_ref, ek_ref, et_ref, tmp_ref, str_ref,
             tw_ref, tb_ref, sw_ref, sb_ref, fw_ref, fb_ref, out_ref):
    f32 = jnp.float32
    fw = fw_ref[...]
    Ws = fw[:, 0:32]
    Wk = fw[:, 32:48]
    Wt = fw[:, 48:64]
    Wtem = fw[:, 64:80]
    Wstr = fw[:, 80:112]

    tW = jnp.dot(tw_ref[...].T, Wtem.T, preferred_element_type=f32)
    sW = jnp.dot(sw_ref[...].T, Wstr.T, preferred_element_type=f32)
    bias = (fb_ref[...]
            + jnp.dot(tb_ref[...], Wtem.T, preferred_element_type=f32)
            + jnp.dot(sb_ref[...], Wstr.T, preferred_element_type=f32))

    out = jnp.dot(es_ref[...], Ws.T, preferred_element_type=f32)
    out += jnp.dot(ek_ref[...], Wk.T, preferred_element_type=f32)
    out += jnp.dot(et_ref[...], Wt.T, preferred_element_type=f32)
    out += jnp.dot(tmp_ref[0], tW, preferred_element_type=f32)
    out += jnp.dot(str_ref[0], sW, preferred_element_type=f32)
    out_ref[...] = out + bias


def _tc_combine(es, ek, et, tempos, structures,
                tempo_W, tempo_b, structure_W, structure_b, final_W, final_b):
    tmp3 = tempos.reshape(_NB, _BLK, 1)
    str3 = structures.reshape(_NB, _BLK, 10)
    tb2 = tempo_b.reshape(1, 16)
    sb2 = structure_b.reshape(1, 32)
    fb2 = final_b.reshape(1, 128)
    full = lambda shape: pl.BlockSpec(shape, lambda i: (0,) * len(shape))
    return pl.pallas_call(
        _tc_body,
        grid=(_NB,),
        in_specs=[
            pl.BlockSpec((_BLK, 32), lambda i: (i, 0)),
            pl.BlockSpec((_BLK, 16), lambda i: (i, 0)),
            pl.BlockSpec((_BLK, 16), lambda i: (i, 0)),
            pl.BlockSpec((1, _BLK, 1), lambda i: (i, 0, 0)),
            pl.BlockSpec((1, _BLK, 10), lambda i: (i, 0, 0)),
            full((16, 1)),
            full((1, 16)),
            full((32, 10)),
            full((1, 32)),
            full((128, 112)),
            full((1, 128)),
        ],
        out_specs=pl.BlockSpec((_BLK, 128), lambda i: (i, 0)),
        out_shape=jax.ShapeDtypeStruct((_B, 128), jnp.float32),
    )(es, ek, et, tmp3, str3, tempo_W, tb2, structure_W, sb2, final_W, fb2)


def kernel(style_ids, key_ids, timefeel_ids, tempos, structures,
           style_table, key_table, timefeel_table,
           tempo_W, tempo_b, structure_W, structure_b,
           final_W, final_b):
    sid = style_ids.astype(jnp.int32).reshape(_B // _C, _C)
    kid = key_ids.astype(jnp.int32).reshape(_B // _C, _C)
    tid = timefeel_ids.astype(jnp.int32).reshape(_B // _C, _C)
    es, ek, et = _sc_gather()(sid, kid, tid,
                              style_table, key_table, timefeel_table)
    return _tc_combine(es, ek, et, tempos, structures,
                       tempo_W, tempo_b, structure_W, structure_b,
                       final_W, final_b)


# speedup vs baseline: 1.0273x; 1.0272x over previous
import functools

import jax
import jax.numpy as jnp
from jax import lax
from jax.experimental import pallas as pl
from jax.experimental.pallas import tpu as pltpu
from jax.experimental.pallas import tpu_sc as plsc

_B = 16384
_BLK = 2048
_NB = _B // _BLK

_NC, _NS, _L = 2, 16, 16
_NW = _NC * _NS
_BPW = _B // _NW


def _sc_body(sid_hbm, kid_hbm, tid_hbm, st_hbm, kt_hbm, tt_hbm, e_hbm,
             sidx_v, kidx_v, tidx_v, st_v, kt_v, tt_v, e_v, sem):
    wid = lax.axis_index("s") * _NC + lax.axis_index("c")
    base = wid * _BPW
    copies = [
        pltpu.async_copy(sid_hbm.at[pl.ds(base, _BPW)], sidx_v, sem),
        pltpu.async_copy(kid_hbm.at[pl.ds(base, _BPW)], kidx_v, sem),
        pltpu.async_copy(tid_hbm.at[pl.ds(base, _BPW)], tidx_v, sem),
        pltpu.async_copy(st_hbm, st_v, sem),
        pltpu.async_copy(kt_hbm, kt_v, sem),
        pltpu.async_copy(tt_hbm, tt_v, sem),
    ]
    for c in copies:
        c.wait()

    it0 = lax.iota(jnp.int32, _L)
    zeros = jnp.zeros((_L,), jnp.float32)

    def group(g, carry):
        off = g * _L
        sid16 = sidx_v[pl.ds(off, _L)]
        kid16 = kidx_v[pl.ds(off, _L)]
        tid16 = tidx_v[pl.ds(off, _L)]
        ridx = it0 + off
        for c in range(32):
            cv = jnp.full((_L,), c, jnp.int32)
            plsc.store_scatter(e_v, [ridx, cv],
                               plsc.load_gather(st_v, [sid16, cv]))
        for c in range(16):
            cv = jnp.full((_L,), c, jnp.int32)
            plsc.store_scatter(e_v, [ridx, jnp.full((_L,), 32 + c, jnp.int32)],
                               plsc.load_gather(kt_v, [kid16, cv]))
            plsc.store_scatter(e_v, [ridx, jnp.full((_L,), 48 + c, jnp.int32)],
                               plsc.load_gather(tt_v, [tid16, cv]))
        for c in range(64, 128):
            plsc.store_scatter(e_v, [ridx, jnp.full((_L,), c, jnp.int32)],
                               zeros)
        return carry

    lax.fori_loop(0, _BPW // _L, group, 0)
    pltpu.sync_copy(e_v, e_hbm.at[pl.ds(base, _BPW)])


@functools.lru_cache(maxsize=1)
def _sc_gather():
    return functools.partial(
        pl.kernel,
        out_type=jax.ShapeDtypeStruct((_B, 128), jnp.float32),
        mesh=plsc.VectorSubcoreMesh(core_axis_name="c", subcore_axis_name="s",
                                    num_cores=_NC, num_subcores=_NS),
        compiler_params=pltpu.CompilerParams(needs_layout_passes=False),
        scratch_types=[
            pltpu.VMEM((_BPW,), jnp.int32),
            pltpu.VMEM((_BPW,), jnp.int32),
            pltpu.VMEM((_BPW,), jnp.int32),
            pltpu.VMEM((50, 32), jnp.float32),
            pltpu.VMEM((24, 16), jnp.float32),
            pltpu.VMEM((20, 16), jnp.float32),
            pltpu.VMEM((_BPW, 128), jnp.float32),
            pltpu.SemaphoreType.DMA,
        ],
    )(_sc_body)


def _tc_body(e_ref, tmp_ref, str_ref,
             tw_ref, tb_ref, sw_ref, sb_ref, fw_ref, fb_ref, out_ref):
    f32 = jnp.float32
    fw = fw_ref[...]
    W64 = fw[:, 0:64]
    Wtem = fw[:, 64:80]
    Wstr = fw[:, 80:112]
    W128 = jnp.concatenate([W64, jnp.zeros((128, 64), f32)], axis=1)

    tW = jnp.dot(tw_ref[...].T, Wtem.T, preferred_element_type=f32)
    sW = jnp.dot(sw_ref[...].T, Wstr.T, preferred_element_type=f32)
    bias = (fb_ref[...]
            + jnp.dot(tb_ref[...], Wtem.T, preferred_element_type=f32)
            + jnp.dot(sb_ref[...], Wstr.T, preferred_element_type=f32))

    out = jnp.dot(e_ref[...], W128.T, preferred_element_type=f32)
    out += jnp.dot(tmp_ref[0], tW, preferred_element_type=f32)
    out += jnp.dot(str_ref[0], sW, preferred_element_type=f32)
    out_ref[...] = out + bias


def _tc_combine(e, tempos, structures,
                tempo_W, tempo_b, structure_W, structure_b, final_W, final_b):
    tmp3 = tempos.reshape(_NB, _BLK, 1)
    str3 = structures.reshape(_NB, _BLK, 10)
    tb2 = tempo_b.reshape(1, 16)
    sb2 = structure_b.reshape(1, 32)
    fb2 = final_b.reshape(1, 128)
    full = lambda shape: pl.BlockSpec(shape, lambda i: (0,) * len(shape))
    return pl.pallas_call(
        _tc_body,
        grid=(_NB,),
        in_specs=[
            pl.BlockSpec((_BLK, 128), lambda i: (i, 0)),
            pl.BlockSpec((1, _BLK, 1), lambda i: (i, 0, 0)),
            pl.BlockSpec((1, _BLK, 10), lambda i: (i, 0, 0)),
            full((16, 1)),
            full((1, 16)),
            full((32, 10)),
            full((1, 32)),
            full((128, 112)),
            full((1, 128)),
        ],
        out_specs=pl.BlockSpec((_BLK, 128), lambda i: (i, 0)),
        out_shape=jax.ShapeDtypeStruct((_B, 128), jnp.float32),
    )(e, tmp3, str3, tempo_W, tb2, structure_W, sb2, final_W, fb2)


def kernel(style_ids, key_ids, timefeel_ids, tempos, structures,
           style_table, key_table, timefeel_table,
           tempo_W, tempo_b, structure_W, structure_b,
           final_W, final_b):
    sid = style_ids.astype(jnp.int32)
    kid = key_ids.astype(jnp.int32)
    tid = timefeel_ids.astype(jnp.int32)
    e = _sc_gather()(sid, kid, tid, style_table, key_table, timefeel_table)
    return _tc_combine(e, tempos, structures,
                       tempo_W, tempo_b, structure_W, structure_b,
                       final_W, final_b)


# speedup vs baseline: 2.2957x; 2.2347x over previous
import functools

import jax
import jax.numpy as jnp
from jax import lax
from jax.experimental import pallas as pl
from jax.experimental.pallas import tpu as pltpu
from jax.experimental.pallas import tpu_sc as plsc

_B = 16384
_BLK = 8192
_NB = _B // _BLK

_NC, _NS, _L = 2, 16, 16
_NW = _NC * _NS
_BPW = _B // _NW


def _sc_body(sid_hbm, kid_hbm, tid_hbm, tbl_hbm, e_hbm,
             sidx_v, kidx_v, tidx_v, tbl_v, e_v, sem, osem):
    wid = lax.axis_index("s") * _NC + lax.axis_index("c")
    base = wid * _BPW
    copies = [
        pltpu.async_copy(sid_hbm.at[pl.ds(base, _BPW)], sidx_v, sem),
        pltpu.async_copy(kid_hbm.at[pl.ds(base, _BPW)], kidx_v, sem),
        pltpu.async_copy(tid_hbm.at[pl.ds(base, _BPW)], tidx_v, sem),
        pltpu.async_copy(tbl_hbm, tbl_v, sem),
    ]
    for c in copies:
        c.wait()

    it0 = lax.iota(jnp.int32, _L)
    it16 = it0 + 16

    _SLAB = 128
    _NSLAB = _BPW // _SLAB
    outs = []
    for s in range(_NSLAB):
        def group(g, carry, s=s):
            off = s * _SLAB + g * _L
            sid16 = sidx_v[pl.ds(off, _L)]
            kid16 = kidx_v[pl.ds(off, _L)] + 50
            tid16 = tidx_v[pl.ds(off, _L)] + 74
            for l in range(_L):
                r = off + l
                sb = jnp.full((_L,), sid16[l], jnp.int32)
                kb = jnp.full((_L,), kid16[l], jnp.int32)
                tb = jnp.full((_L,), tid16[l], jnp.int32)
                e_v[r, pl.ds(0, _L)] = plsc.load_gather(tbl_v, [sb, it0])
                e_v[r, pl.ds(16, _L)] = plsc.load_gather(tbl_v, [sb, it16])
                e_v[r, pl.ds(32, _L)] = plsc.load_gather(tbl_v, [kb, it0])
                e_v[r, pl.ds(48, _L)] = plsc.load_gather(tbl_v, [tb, it0])
            return carry

        lax.fori_loop(0, _SLAB // _L, group, 0)
        outs.append(pltpu.async_copy(
            e_v.at[pl.ds(s * _SLAB, _SLAB)],
            e_hbm.at[pl.ds(base + s * _SLAB, _SLAB)], osem))
    for c in outs:
        c.wait()


@functools.lru_cache(maxsize=1)
def _sc_gather():
    return functools.partial(
        pl.kernel,
        out_type=jax.ShapeDtypeStruct((_B, 64), jnp.float32),
        mesh=plsc.VectorSubcoreMesh(core_axis_name="c", subcore_axis_name="s",
                                    num_cores=_NC, num_subcores=_NS),
        compiler_params=pltpu.CompilerParams(needs_layout_passes=False),
        scratch_types=[
            pltpu.VMEM((_BPW,), jnp.int32),
            pltpu.VMEM((_BPW,), jnp.int32),
            pltpu.VMEM((_BPW,), jnp.int32),
            pltpu.VMEM((94, 32), jnp.float32),
            pltpu.VMEM((_BPW, 64), jnp.float32),
            pltpu.SemaphoreType.DMA,
            pltpu.SemaphoreType.DMA,
        ],
    )(_sc_body)


def _tc_body(e_ref, tmp_ref, str_ref,
             tw_ref, tb_ref, sw_ref, sb_ref, fw_ref, fb_ref, out_ref):
    f32 = jnp.float32
    fw = fw_ref[...]
    W64 = fw[:, 0:64]
    Wtem = fw[:, 64:80]
    Wstr = fw[:, 80:112]

    tW = jnp.dot(tw_ref[...].T, Wtem.T, preferred_element_type=f32)
    sW = jnp.dot(sw_ref[...].T, Wstr.T, preferred_element_type=f32)
    bias = (fb_ref[...]
            + jnp.dot(tb_ref[...], Wtem.T, preferred_element_type=f32)
            + jnp.dot(sb_ref[...], Wstr.T, preferred_element_type=f32))

    out = jnp.dot(e_ref[...], W64.T, preferred_element_type=f32)
    out += lax.dot_general(str_ref[...], sW, (((0,), (0,)), ((), ())),
                           preferred_element_type=f32)
    out += tmp_ref[0, 0, :][:, None] * tW
    out_ref[...] = out + bias


def _tc_combine(e, tempos, structures_t,
                tempo_W, tempo_b, structure_W, structure_b, final_W, final_b):
    tmp3 = tempos.reshape(_NB, 1, _BLK)
    tb2 = tempo_b.reshape(1, 16)
    sb2 = structure_b.reshape(1, 32)
    fb2 = final_b.reshape(1, 128)
    full = lambda shape: pl.BlockSpec(shape, lambda i: (0,) * len(shape))
    return pl.pallas_call(
        _tc_body,
        grid=(_NB,),
        in_specs=[
            pl.BlockSpec((_BLK, 64), lambda i: (i, 0)),
            pl.BlockSpec((1, 1, _BLK), lambda i: (i, 0, 0)),
            pl.BlockSpec((10, _BLK), lambda i: (0, i)),
            full((16, 1)),
            full((1, 16)),
            full((32, 10)),
            full((1, 32)),
            full((128, 112)),
            full((1, 128)),
        ],
        out_specs=pl.BlockSpec((_BLK, 128), lambda i: (i, 0)),
        out_shape=jax.ShapeDtypeStruct((_B, 128), jnp.float32),
    )(e, tmp3, structures_t, tempo_W, tb2, structure_W, sb2, final_W, fb2)


def kernel(style_ids, key_ids, timefeel_ids, tempos, structures,
           style_table, key_table, timefeel_table,
           tempo_W, tempo_b, structure_W, structure_b,
           final_W, final_b):
    sid = style_ids.astype(jnp.int32)
    kid = key_ids.astype(jnp.int32)
    tid = timefeel_ids.astype(jnp.int32)
    tbl = jnp.concatenate([style_table,
                           jnp.pad(key_table, ((0, 0), (0, 16))),
                           jnp.pad(timefeel_table, ((0, 0), (0, 16)))])
    e = _sc_gather()(sid, kid, tid, tbl)
    return _tc_combine(e, tempos.reshape(_B), structures.T,
                       tempo_W, tempo_b, structure_W, structure_b,
                       final_W, final_b)
